# baseline (speedup 1.0000x reference)
import jax
import jax.numpy as jnp
from jax.experimental import pallas as pl
from jax.experimental.pallas import tpu as pltpu

_NCLS = 21
_TOPK = 200
_CONF_THRESH = 0.01
_NMS_THRESH = 0.45
_VAR0, _VAR1 = 0.1, 0.2

_P = 8732
_PPAD = 8832
_CPAD = 24
_KPAD = 256


def _ssd_kernel(conf_ref, loc_ref, pri_ref, o_ref,
                s_ref, x1_ref, y1_ref, x2_ref, y2_ref, ar_ref, va_ref):
    c = conf_ref[0]
    mx = jnp.max(c, axis=0, keepdims=True)
    e = jnp.exp(c - mx)
    p = e / jnp.sum(e, axis=0, keepdims=True)

    lane = jax.lax.broadcasted_iota(jnp.int32, (_CPAD, _PPAD), 1)
    row = jax.lax.broadcasted_iota(jnp.int32, (_CPAD, _PPAD), 0)
    active = (lane < _P) & (row >= 1) & (row <= _NCLS - 1)
    s_ref[...] = jnp.where(active, p, 0.0)

    lc = loc_ref[0]
    pr = pri_ref[...]
    cx = pr[0:1] + lc[0:1] * _VAR0 * pr[2:3]
    cy = pr[1:2] + lc[1:2] * _VAR0 * pr[3:4]
    w = pr[2:3] * jnp.exp(lc[2:3] * _VAR1)
    h = pr[3:4] * jnp.exp(lc[3:4] * _VAR1)
    x1 = cx - w * 0.5
    y1 = cy - h * 0.5
    x2 = cx + w * 0.5
    y2 = cy + h * 0.5
    area = (x2 - x1) * (y2 - y1)
    zrow = jnp.zeros_like(x1)
    boxes8 = jnp.concatenate([x1, y1, x2, y2, area, zrow, zrow, zrow], axis=0)

    for r in (x1_ref, y1_ref, x2_ref, y2_ref, ar_ref, va_ref):
        r[...] = jnp.zeros_like(r[...])

    kiota = jax.lax.broadcasted_iota(jnp.int32, (_CPAD, _KPAD), 1)

    def body(_, cnt):
        s = s_ref[...]
        m = jnp.max(s, axis=1, keepdims=True)
        idx = jnp.min(jnp.where(s == m, lane, _PPAD), axis=1, keepdims=True)
        oh = lane == idx
        s_ref[...] = jnp.where(oh, -1.0, s)
        bx = jax.lax.dot_general(oh.astype(jnp.float32), boxes8,
                                 (((1,), (1,)), ((), ())),
                                 precision=jax.lax.Precision.HIGHEST)
        bx1 = bx[:, 0:1]
        by1 = bx[:, 1:2]
        bx2 = bx[:, 2:3]
        by2 = bx[:, 3:4]
        bar = bx[:, 4:5]
        kx1 = x1_ref[...]
        ky1 = y1_ref[...]
        kx2 = x2_ref[...]
        ky2 = y2_ref[...]
        kar = ar_ref[...]
        iw = jnp.maximum(jnp.minimum(bx2, kx2) - jnp.maximum(bx1, kx1), 0.0)
        ih = jnp.maximum(jnp.minimum(by2, ky2) - jnp.maximum(by1, ky1), 0.0)
        inter = iw * ih
        iou = inter / (bar + kar - inter)
        live = kiota < cnt
        sup = jnp.any(live & (iou > _NMS_THRESH), axis=1, keepdims=True)
        keep = (m > _CONF_THRESH) & jnp.logical_not(sup)
        poh = (kiota == cnt) & keep
        va_ref[...] = jnp.where(poh, m, va_ref[...])
        x1_ref[...] = jnp.where(poh, bx1, kx1)
        y1_ref[...] = jnp.where(poh, by1, ky1)
        x2_ref[...] = jnp.where(poh, bx2, kx2)
        y2_ref[...] = jnp.where(poh, by2, ky2)
        ar_ref[...] = jnp.where(poh, bar, kar)
        return cnt + keep.astype(jnp.int32)

    jax.lax.fori_loop(0, _TOPK, body, jnp.zeros((_CPAD, 1), jnp.int32))

    o_ref[0, 0] = va_ref[...]
    o_ref[0, 1] = x1_ref[...]
    o_ref[0, 2] = y1_ref[...]
    o_ref[0, 3] = x2_ref[...]
    o_ref[0, 4] = y2_ref[...]
    zk = jnp.zeros((_CPAD, _KPAD), jnp.float32)
    o_ref[0, 5] = zk
    o_ref[0, 6] = zk
    o_ref[0, 7] = zk


@jax.jit
def _run(loc_data, conf_data, prior_data):
    B = loc_data.shape[0]
    conf_t = jnp.pad(jnp.transpose(conf_data, (0, 2, 1)),
                     ((0, 0), (0, _CPAD - _NCLS), (0, _PPAD - _P)),
                     constant_values=-1e9)
    loc_t = jnp.pad(jnp.transpose(loc_data, (0, 2, 1)),
                    ((0, 0), (0, 4), (0, _PPAD - _P)))
    pri_t = jnp.pad(jnp.transpose(prior_data, (1, 0)),
                    ((0, 4), (0, _PPAD - _P)))
    out = pl.pallas_call(
        _ssd_kernel,
        grid=(B,),
        in_specs=[
            pl.BlockSpec((1, _CPAD, _PPAD), lambda b: (b, 0, 0)),
            pl.BlockSpec((1, 8, _PPAD), lambda b: (b, 0, 0)),
            pl.BlockSpec((8, _PPAD), lambda b: (0, 0)),
        ],
        out_specs=pl.BlockSpec((1, 8, _CPAD, _KPAD), lambda b: (b, 0, 0, 0)),
        out_shape=jax.ShapeDtypeStruct((B, 8, _CPAD, _KPAD), jnp.float32),
        scratch_shapes=[pltpu.VMEM((_CPAD, _PPAD), jnp.float32)]
        + [pltpu.VMEM((_CPAD, _KPAD), jnp.float32)] * 6,
        compiler_params=pltpu.CompilerParams(
            dimension_semantics=("parallel",)),
    )(conf_t, loc_t, pri_t)
    return jnp.transpose(out[:, 0:5, 0:_NCLS, 0:_TOPK], (0, 2, 3, 1))


def kernel(loc_data, conf_data, prior_data):
    return _run(loc_data, conf_data, prior_data)


# speedup vs baseline: 1.3139x; 1.3139x over previous
import jax
import jax.numpy as jnp
from jax.experimental import pallas as pl
from jax.experimental.pallas import tpu as pltpu

_NCLS = 21
_TOPK = 200
_CONF_THRESH = 0.01
_NMS_THRESH = 0.45
_VAR0, _VAR1 = 0.1, 0.2

_P = 8732
_PPAD = 8832
_CPAD = 24
_KPAD = 256


def _ssd_kernel(conf_ref, loc_ref, pri_ref, o_ref,
                s_ref, x1_ref, y1_ref, x2_ref, y2_ref, ar_ref, va_ref):
    c = conf_ref[0]
    mx = jnp.max(c, axis=0, keepdims=True)
    e = jnp.exp(c - mx)
    p = e / jnp.sum(e, axis=0, keepdims=True)

    lane = jax.lax.broadcasted_iota(jnp.int32, (_CPAD, _PPAD), 1)
    row = jax.lax.broadcasted_iota(jnp.int32, (_CPAD, _PPAD), 0)
    active = (lane < _P) & (row >= 1) & (row <= _NCLS - 1)
    s_ref[...] = jnp.where(active, p, 0.0)

    lc = loc_ref[0]
    pr = pri_ref[...]
    cx = pr[0:1] + lc[0:1] * _VAR0 * pr[2:3]
    cy = pr[1:2] + lc[1:2] * _VAR0 * pr[3:4]
    w = pr[2:3] * jnp.exp(lc[2:3] * _VAR1)
    h = pr[3:4] * jnp.exp(lc[3:4] * _VAR1)
    x1 = cx - w * 0.5
    y1 = cy - h * 0.5
    x2 = cx + w * 0.5
    y2 = cy + h * 0.5
    area = (x2 - x1) * (y2 - y1)
    zrow = jnp.zeros_like(x1)
    boxes8 = jnp.concatenate([x1, y1, x2, y2, area, zrow, zrow, zrow], axis=0)
    b_hi = boxes8.astype(jnp.bfloat16)
    r1 = boxes8 - b_hi.astype(jnp.float32)
    b_mid = r1.astype(jnp.bfloat16)
    b_lo = (r1 - b_mid.astype(jnp.float32)).astype(jnp.bfloat16)

    for r in (x1_ref, y1_ref, x2_ref, y2_ref, ar_ref, va_ref):
        r[...] = jnp.zeros_like(r[...])

    kiota = jax.lax.broadcasted_iota(jnp.int32, (_CPAD, _KPAD), 1)

    def body(_, cnt):
        s = s_ref[...]
        m = jnp.max(s, axis=1, keepdims=True)
        idx = jnp.min(jnp.where(s == m, lane, _PPAD), axis=1, keepdims=True)
        oh = lane == idx
        s_ref[...] = jnp.where(oh, -1.0, s)
        ohb = jnp.where(oh, 1.0, 0.0).astype(jnp.bfloat16)
        dn = (((1,), (1,)), ((), ()))
        bx = (jax.lax.dot_general(ohb, b_hi, dn,
                                  preferred_element_type=jnp.float32)
              + jax.lax.dot_general(ohb, b_mid, dn,
                                    preferred_element_type=jnp.float32)
              + jax.lax.dot_general(ohb, b_lo, dn,
                                    preferred_element_type=jnp.float32))
        bx1 = bx[:, 0:1]
        by1 = bx[:, 1:2]
        bx2 = bx[:, 2:3]
        by2 = bx[:, 3:4]
        bar = bx[:, 4:5]
        kx1 = x1_ref[...]
        ky1 = y1_ref[...]
        kx2 = x2_ref[...]
        ky2 = y2_ref[...]
        kar = ar_ref[...]
        iw = jnp.maximum(jnp.minimum(bx2, kx2) - jnp.maximum(bx1, kx1), 0.0)
        ih = jnp.maximum(jnp.minimum(by2, ky2) - jnp.maximum(by1, ky1), 0.0)
        inter = iw * ih
        iou = inter / (bar + kar - inter)
        live = kiota < cnt
        sup = jnp.any(live & (iou > _NMS_THRESH), axis=1, keepdims=True)
        keep = (m > _CONF_THRESH) & jnp.logical_not(sup)
        poh = (kiota == cnt) & keep
        va_ref[...] = jnp.where(poh, m, va_ref[...])
        x1_ref[...] = jnp.where(poh, bx1, kx1)
        y1_ref[...] = jnp.where(poh, by1, ky1)
        x2_ref[...] = jnp.where(poh, bx2, kx2)
        y2_ref[...] = jnp.where(poh, by2, ky2)
        ar_ref[...] = jnp.where(poh, bar, kar)
        return cnt + keep.astype(jnp.int32)

    jax.lax.fori_loop(0, _TOPK, body, jnp.zeros((_CPAD, 1), jnp.int32))

    o_ref[0, 0] = va_ref[...]
    o_ref[0, 1] = x1_ref[...]
    o_ref[0, 2] = y1_ref[...]
    o_ref[0, 3] = x2_ref[...]
    o_ref[0, 4] = y2_ref[...]
    zk = jnp.zeros((_CPAD, _KPAD), jnp.float32)
    o_ref[0, 5] = zk
    o_ref[0, 6] = zk
    o_ref[0, 7] = zk


@jax.jit
def _run(loc_data, conf_data, prior_data):
    B = loc_data.shape[0]
    conf_t = jnp.pad(jnp.transpose(conf_data, (0, 2, 1)),
                     ((0, 0), (0, _CPAD - _NCLS), (0, _PPAD - _P)),
                     constant_values=-1e9)
    loc_t = jnp.pad(jnp.transpose(loc_data, (0, 2, 1)),
                    ((0, 0), (0, 4), (0, _PPAD - _P)))
    pri_t = jnp.pad(jnp.transpose(prior_data, (1, 0)),
                    ((0, 4), (0, _PPAD - _P)))
    out = pl.pallas_call(
        _ssd_kernel,
        grid=(B,),
        in_specs=[
            pl.BlockSpec((1, _CPAD, _PPAD), lambda b: (b, 0, 0)),
            pl.BlockSpec((1, 8, _PPAD), lambda b: (b, 0, 0)),
            pl.BlockSpec((8, _PPAD), lambda b: (0, 0)),
        ],
        out_specs=pl.BlockSpec((1, 8, _CPAD, _KPAD), lambda b: (b, 0, 0, 0)),
        out_shape=jax.ShapeDtypeStruct((B, 8, _CPAD, _KPAD), jnp.float32),
        scratch_shapes=[pltpu.VMEM((_CPAD, _PPAD), jnp.float32)]
        + [pltpu.VMEM((_CPAD, _KPAD), jnp.float32)] * 6,
        compiler_params=pltpu.CompilerParams(
            dimension_semantics=("parallel",)),
    )(conf_t, loc_t, pri_t)
    return jnp.transpose(out[:, 0:5, 0:_NCLS, 0:_TOPK], (0, 2, 3, 1))


def kernel(loc_data, conf_data, prior_data):
    return _run(loc_data, conf_data, prior_data)


# speedup vs baseline: 3.3090x; 2.5185x over previous
import jax
import jax.numpy as jnp
from jax.experimental import pallas as pl
from jax.experimental.pallas import tpu as pltpu

_NCLS = 21
_TOPK = 200
_CONF_THRESH = 0.01
_NMS_THRESH = 0.45
_VAR0, _VAR1 = 0.1, 0.2

_P = 8732
_PPAD = 8832
_CPAD = 24
_KPAD = 256


def _ssd_kernel(conf_ref, loc_ref, pri_ref, o_ref,
                s_ref, x1_ref, y1_ref, x2_ref, y2_ref, ar_ref, va_ref):
    c = conf_ref[0]
    mx = jnp.max(c, axis=0, keepdims=True)
    e = jnp.exp(c - mx)
    p = e / jnp.sum(e, axis=0, keepdims=True)

    lane = jax.lax.broadcasted_iota(jnp.int32, (_CPAD, _PPAD), 1)
    row = jax.lax.broadcasted_iota(jnp.int32, (_CPAD, _PPAD), 0)
    active = (lane < _P) & (row >= 1) & (row <= _NCLS - 1)
    s_ref[...] = jnp.where(active, p, 0.0)

    lc = loc_ref[0]
    pr = pri_ref[...]
    cx = pr[0:1] + lc[0:1] * _VAR0 * pr[2:3]
    cy = pr[1:2] + lc[1:2] * _VAR0 * pr[3:4]
    w = pr[2:3] * jnp.exp(lc[2:3] * _VAR1)
    h = pr[3:4] * jnp.exp(lc[3:4] * _VAR1)
    x1 = cx - w * 0.5
    y1 = cy - h * 0.5
    x2 = cx + w * 0.5
    y2 = cy + h * 0.5
    area = (x2 - x1) * (y2 - y1)
    zrow = jnp.zeros_like(x1)
    boxes8 = jnp.concatenate([x1, y1, x2, y2, area, zrow, zrow, zrow], axis=0)
    b_hi = boxes8.astype(jnp.bfloat16)
    r1 = boxes8 - b_hi.astype(jnp.float32)
    b_mid = r1.astype(jnp.bfloat16)
    b_lo = (r1 - b_mid.astype(jnp.float32)).astype(jnp.bfloat16)
    b_all = jnp.concatenate([b_hi, b_mid, b_lo], axis=0)

    for r in (x1_ref, y1_ref, x2_ref, y2_ref, ar_ref, va_ref):
        r[...] = jnp.zeros_like(r[...])

    kiota = jax.lax.broadcasted_iota(jnp.int32, (_CPAD, _KPAD), 1)

    def body(_, cnt):
        s = s_ref[...]
        m = jnp.max(s, axis=1, keepdims=True)
        idx = jnp.argmax(s, axis=1, keepdims=True).astype(jnp.int32)
        oh = lane == idx
        s_ref[...] = jnp.where(oh, -1.0, s)
        ohb = jnp.where(oh, 1.0, 0.0).astype(jnp.bfloat16)
        dn = (((1,), (1,)), ((), ()))
        bx3 = jax.lax.dot_general(ohb, b_all, dn,
                                  preferred_element_type=jnp.float32)
        bx = bx3[:, 0:8] + bx3[:, 8:16] + bx3[:, 16:24]
        bx1 = bx[:, 0:1]
        by1 = bx[:, 1:2]
        bx2 = bx[:, 2:3]
        by2 = bx[:, 3:4]
        bar = bx[:, 4:5]
        kx1 = x1_ref[...]
        ky1 = y1_ref[...]
        kx2 = x2_ref[...]
        ky2 = y2_ref[...]
        kar = ar_ref[...]
        iw = jnp.maximum(jnp.minimum(bx2, kx2) - jnp.maximum(bx1, kx1), 0.0)
        ih = jnp.maximum(jnp.minimum(by2, ky2) - jnp.maximum(by1, ky1), 0.0)
        inter = iw * ih
        iou = inter / (bar + kar - inter)
        sup = jnp.any(iou > _NMS_THRESH, axis=1, keepdims=True)
        keep = (m > _CONF_THRESH) & jnp.logical_not(sup)
        poh = (kiota == cnt) & keep
        va_ref[...] = jnp.where(poh, m, va_ref[...])
        x1_ref[...] = jnp.where(poh, bx1, kx1)
        y1_ref[...] = jnp.where(poh, by1, ky1)
        x2_ref[...] = jnp.where(poh, bx2, kx2)
        y2_ref[...] = jnp.where(poh, by2, ky2)
        ar_ref[...] = jnp.where(poh, bar, kar)
        return cnt + keep.astype(jnp.int32)

    jax.lax.fori_loop(0, _TOPK, body, jnp.zeros((_CPAD, 1), jnp.int32),
                      unroll=8)

    o_ref[0, 0] = va_ref[...]
    o_ref[0, 1] = x1_ref[...]
    o_ref[0, 2] = y1_ref[...]
    o_ref[0, 3] = x2_ref[...]
    o_ref[0, 4] = y2_ref[...]
    zk = jnp.zeros((_CPAD, _KPAD), jnp.float32)
    o_ref[0, 5] = zk
    o_ref[0, 6] = zk
    o_ref[0, 7] = zk


@jax.jit
def _run(loc_data, conf_data, prior_data):
    B = loc_data.shape[0]
    conf_t = jnp.pad(jnp.transpose(conf_data, (0, 2, 1)),
                     ((0, 0), (0, _CPAD - _NCLS), (0, _PPAD - _P)),
                     constant_values=-1e9)
    loc_t = jnp.pad(jnp.transpose(loc_data, (0, 2, 1)),
                    ((0, 0), (0, 4), (0, _PPAD - _P)))
    pri_t = jnp.pad(jnp.transpose(prior_data, (1, 0)),
                    ((0, 4), (0, _PPAD - _P)))
    out = pl.pallas_call(
        _ssd_kernel,
        grid=(B,),
        in_specs=[
            pl.BlockSpec((1, _CPAD, _PPAD), lambda b: (b, 0, 0)),
            pl.BlockSpec((1, 8, _PPAD), lambda b: (b, 0, 0)),
            pl.BlockSpec((8, _PPAD), lambda b: (0, 0)),
        ],
        out_specs=pl.BlockSpec((1, 8, _CPAD, _KPAD), lambda b: (b, 0, 0, 0)),
        out_shape=jax.ShapeDtypeStruct((B, 8, _CPAD, _KPAD), jnp.float32),
        scratch_shapes=[pltpu.VMEM((_CPAD, _PPAD), jnp.float32)]
        + [pltpu.VMEM((_CPAD, _KPAD), jnp.float32)] * 6,
        compiler_params=pltpu.CompilerParams(
            dimension_semantics=("parallel",)),
    )(conf_t, loc_t, pri_t)
    return jnp.transpose(out[:, 0:5, 0:_NCLS, 0:_TOPK], (0, 2, 3, 1))


def kernel(loc_data, conf_data, prior_data):
    return _run(loc_data, conf_data, prior_data)
